# baseline (speedup 1.0000x reference)
import functools

import jax
import jax.numpy as jnp
from jax import lax
from jax.experimental import pallas as pl
from jax.experimental.pallas import tpu as pltpu
from jax.experimental.pallas import tpu_sc as plsc

_THETA = 0.25
_L = 16
_NC = 2
_NS = 16
_NT = 5
_NW = _NC * _NS


def _row0_body(pair_ref, out_ref):
    out_ref[...] = pair_ref[0]


def _extract_row0(edgeij_pair):
    E = edgeij_pair.shape[1]
    return pl.pallas_call(
        _row0_body,
        out_shape=jax.ShapeDtypeStruct((E,), edgeij_pair.dtype),
    )(edgeij_pair)


@functools.partial(jax.jit, static_argnums=(2,))
def _segmin_edge_update(pair, edge_attr, n_nodes):
    row = _extract_row0(pair)
    E = pair.shape[1]
    grain = _L * _NT * _NW
    e1 = ((E // 2) // grain) * grain + grain
    e2 = E - e1
    assert e2 % grain == 0
    ept1 = e1 // _NW
    ept2 = e2 // _NW
    npad = ((n_nodes + _L * _NS - 1) // (_L * _NS)) * (_L * _NS)
    chunk = npad // _NS

    a1 = edge_attr[:e1, 0]
    a2 = edge_attr[e1:, 0]

    mesh = plsc.VectorSubcoreMesh(core_axis_name="c", subcore_axis_name="s")
    cparams = pltpu.CompilerParams(needs_layout_passes=False)

    def sweeps(row_v, a_v, tabs, n_edges):
        trips = n_edges // (_L * _NT)

        def sweep(_):
            def body(i, acc):
                b = i * _NT * _L
                idxs = [row_v[pl.ds(b + u * _L, _L)] for u in range(_NT)]
                avs = [a_v[pl.ds(b + u * _L, _L)] for u in range(_NT)]
                curs = [plsc.load_gather(tabs[u], [idxs[u]])
                        for u in range(_NT)]
                losts = [avs[u] < curs[u] for u in range(_NT)]
                for u in range(_NT):
                    plsc.store_scatter(tabs[u], [idxs[u]], avs[u],
                                       mask=losts[u])
                chks = [plsc.load_gather(tabs[u], [idxs[u]])
                        for u in range(_NT)]
                for u in range(_NT):
                    acc = acc | (avs[u] < chks[u])
                return acc
            return lax.fori_loop(0, trips, body, jnp.zeros((_L,), jnp.bool_))

        fail = sweep(0)
        lax.while_loop(lambda f: jnp.any(f), sweep, fail)

    def init_tables(tabs):
        inf16 = jnp.full((_L,), jnp.inf, jnp.float32)

        def init_body(i, c):
            for t in tabs:
                t[pl.ds(i * _L, _L)] = inf16
            return c
        lax.fori_loop(0, npad // _L, init_body, 0)

    @functools.partial(
        pl.kernel,
        out_type=jax.ShapeDtypeStruct((_NW * npad,), jnp.float32),
        mesh=mesh,
        compiler_params=cparams,
        scratch_types=[
            pltpu.VMEM((ept1,), jnp.int32),
            pltpu.VMEM((ept1,), jnp.float32),
            [pltpu.VMEM((npad,), jnp.float32) for _ in range(_NT)],
            pltpu.VMEM((npad,), jnp.float32),
            pltpu.SemaphoreType.DMA,
            pltpu.SemaphoreType.DMA,
        ],
    )
    def k1a(row_hbm, a_hbm, tdump_hbm, row_v, a_v, tabs, tabm, sem1, sem2):
        cid = lax.axis_index("c")
        sid = lax.axis_index("s")
        wid = cid * _NS + sid
        base = wid * ept1

        cp_row = pltpu.async_copy(row_hbm.at[pl.ds(base, ept1)], row_v, sem1)
        cp_a = pltpu.async_copy(a_hbm.at[pl.ds(base, ept1)], a_v, sem2)
        init_tables(tabs)
        cp_row.wait()
        cp_a.wait()

        sweeps(row_v, a_v, tabs, ept1)

        def merge_body(j, c):
            jo = j * _L
            m0 = tabs[0][pl.ds(jo, _L)]
            for t in tabs[1:]:
                m0 = jnp.minimum(m0, t[pl.ds(jo, _L)])
            tabm[pl.ds(jo, _L)] = m0
            return c
        lax.fori_loop(0, npad // _L, merge_body, 0)

        pltpu.sync_copy(tabm, tdump_hbm.at[pl.ds(wid * npad, npad)])

    @functools.partial(
        pl.kernel,
        out_type=jax.ShapeDtypeStruct((_NC * npad,), jnp.float32),
        mesh=mesh,
        compiler_params=cparams,
        scratch_types=[
            pltpu.VMEM((ept2,), jnp.int32),
            pltpu.VMEM((ept2,), jnp.float32),
            [pltpu.VMEM((npad,), jnp.float32) for _ in range(_NT)],
            pltpu.VMEM((npad,), jnp.float32),
            pltpu.VMEM((npad,), jnp.float32),
            pltpu.VMEM_SHARED((_NS, npad), jnp.float32),
            pltpu.SemaphoreType.DMA,
            pltpu.SemaphoreType.DMA,
        ],
    )
    def k1b(row_hbm, a_hbm, tdump_hbm, tab_hbm, row_v, a_v, tabs, prev_v,
            tabm, sp_tab, sem1, sem2):
        cid = lax.axis_index("c")
        sid = lax.axis_index("s")
        wid = cid * _NS + sid
        base = e1 + wid * ept2

        cp_row = pltpu.async_copy(row_hbm.at[pl.ds(base, ept2)], row_v, sem1)
        cp_a = pltpu.async_copy(a_hbm.at[pl.ds(wid * ept2, ept2)], a_v, sem2)
        cp_prev = pltpu.async_copy(tdump_hbm.at[pl.ds(wid * npad, npad)],
                                   prev_v, sem1)
        init_tables(tabs)
        cp_row.wait()
        cp_a.wait()

        sweeps(row_v, a_v, tabs, ept2)
        cp_prev.wait()

        def merge_body(j, c):
            jo = j * _L
            m0 = prev_v[pl.ds(jo, _L)]
            for t in tabs:
                m0 = jnp.minimum(m0, t[pl.ds(jo, _L)])
            tabm[pl.ds(jo, _L)] = m0
            return c
        lax.fori_loop(0, npad // _L, merge_body, 0)

        pltpu.sync_copy(tabm, sp_tab.at[sid])
        plsc.subcore_barrier()

        cb = sid * chunk
        stage = tabs[0]
        descs = [pltpu.async_copy(sp_tab.at[r, pl.ds(cb, chunk)],
                                  stage.at[pl.ds(r * chunk, chunk)], sem1)
                 for r in range(_NS)]
        for d in descs:
            d.wait()

        res = tabs[1]

        def red_body(j, c):
            jo = j * _L
            m0 = stage[pl.ds(jo, _L)]
            for r in range(1, _NS):
                m0 = jnp.minimum(m0, stage[pl.ds(r * chunk + jo, _L)])
            res[pl.ds(cb + jo, _L)] = m0
            return c
        lax.fori_loop(0, chunk // _L, red_body, 0)

        pltpu.sync_copy(res.at[pl.ds(cb, chunk)],
                        tab_hbm.at[pl.ds(cid * npad + cb, chunk)])

    @functools.partial(
        pl.kernel,
        out_type=jax.ShapeDtypeStruct((E,), jnp.float32),
        mesh=mesh,
        compiler_params=cparams,
        scratch_types=[
            pltpu.VMEM((_NC * npad,), jnp.float32),
            pltpu.VMEM((npad,), jnp.float32),
            pltpu.VMEM((ept1,), jnp.int32),
            pltpu.VMEM((ept1,), jnp.float32),
            pltpu.VMEM((ept1,), jnp.float32),
            pltpu.SemaphoreType.DMA,
            pltpu.SemaphoreType.DMA,
        ],
    )
    def k2(tab_hbm, row_hbm, a1_hbm, a2_hbm, out_hbm, s2_v, tabm, row_v,
           a_v, w_v, sem1, sem2):
        cid = lax.axis_index("c")
        sid = lax.axis_index("s")
        wid = cid * _NS + sid

        cp_tab = pltpu.async_copy(tab_hbm, s2_v, sem1)
        cp_tab.wait()

        def merge_body(j, c):
            jo = j * _L
            tabm[pl.ds(jo, _L)] = jnp.minimum(s2_v[pl.ds(jo, _L)],
                                              s2_v[pl.ds(npad + jo, _L)])
            return c
        lax.fori_loop(0, npad // _L, merge_body, 0)

        for (a_hbm, aoff, goff, n_edges) in (
                (a1_hbm, wid * ept1, wid * ept1, ept1),
                (a2_hbm, wid * ept2, e1 + wid * ept2, ept2)):
            cp_row = pltpu.async_copy(row_hbm.at[pl.ds(goff, n_edges)],
                                      row_v.at[pl.ds(0, n_edges)], sem2)
            cp_a = pltpu.async_copy(a_hbm.at[pl.ds(aoff, n_edges)],
                                    a_v.at[pl.ds(0, n_edges)], sem2)
            cp_row.wait()
            cp_a.wait()

            def p2_body(j, c):
                b = j * _NT * _L
                idxs = [row_v[pl.ds(b + u * _L, _L)] for u in range(_NT)]
                avs = [a_v[pl.ds(b + u * _L, _L)] for u in range(_NT)]
                ms = [plsc.load_gather(tabm, [idxs[u]]) for u in range(_NT)]
                for u in range(_NT):
                    w_v[pl.ds(b + u * _L, _L)] = jnp.maximum(
                        avs[u] / ms[u] - _THETA, 0.0)
                return c
            lax.fori_loop(0, n_edges // (_L * _NT), p2_body, 0)

            pltpu.sync_copy(w_v.at[pl.ds(0, n_edges)],
                            out_hbm.at[pl.ds(goff, n_edges)])

    tdump = k1a(row, a1)
    tab2 = k1b(row, a2, tdump)
    return k2(tab2, row, a1, a2)


def kernel(vertex_attr, edgeij_pair, edge_attr):
    return _segmin_edge_update(edgeij_pair, edge_attr, vertex_attr.shape[0])


# speedup vs baseline: 1.1374x; 1.1374x over previous
import functools

import jax
import jax.numpy as jnp
from jax import lax
from jax.experimental import pallas as pl
from jax.experimental.pallas import tpu as pltpu
from jax.experimental.pallas import tpu_sc as plsc

_THETA = 0.25
_L = 16
_NC = 2
_NS = 16
_NT = 5
_NW = _NC * _NS


def _row0_body(pair_ref, out_ref):
    out_ref[...] = pair_ref[0]


def _extract_row0(edgeij_pair):
    E = edgeij_pair.shape[1]
    return pl.pallas_call(
        _row0_body,
        out_shape=jax.ShapeDtypeStruct((E,), edgeij_pair.dtype),
    )(edgeij_pair)


@functools.partial(jax.jit, static_argnums=(2,))
def _segmin_edge_update(pair, edge_attr, n_nodes):
    row = _extract_row0(pair)
    E = pair.shape[1]
    grain = _L * _NT * _NW
    e1 = ((E // 2) // grain) * grain + grain
    e2 = E - e1
    assert e2 % grain == 0
    ept1 = e1 // _NW
    ept2 = e2 // _NW
    npad = ((n_nodes + _L * _NS - 1) // (_L * _NS)) * (_L * _NS)
    chunk = npad // _NS

    a1 = edge_attr[:e1, 0]
    a2 = edge_attr[e1:, 0]

    mesh = plsc.VectorSubcoreMesh(core_axis_name="c", subcore_axis_name="s")
    cparams = pltpu.CompilerParams(needs_layout_passes=False)

    @functools.partial(
        pl.kernel,
        out_type=jax.ShapeDtypeStruct((_NC * npad,), jnp.float32),
        mesh=mesh,
        compiler_params=cparams,
        scratch_types=[
            pltpu.VMEM((ept1 + ept2,), jnp.int32),
            pltpu.VMEM((ept1 + ept2,), jnp.float32),
            [pltpu.VMEM((npad,), jnp.float32) for _ in range(_NT)],
            pltpu.VMEM((npad,), jnp.float32),
            pltpu.VMEM_SHARED((_NS, npad), jnp.float32),
            pltpu.SemaphoreType.DMA,
            pltpu.SemaphoreType.DMA,
        ],
    )
    def k1(row_hbm, a1_hbm, a2_hbm, tab_hbm, row_v, a_v, tabs, tabm,
           sp_tab, sem1, sem2):
        cid = lax.axis_index("c")
        sid = lax.axis_index("s")
        wid = cid * _NS + sid

        cps = [
            pltpu.async_copy(row_hbm.at[pl.ds(wid * ept1, ept1)],
                             row_v.at[pl.ds(0, ept1)], sem1),
            pltpu.async_copy(row_hbm.at[pl.ds(e1 + wid * ept2, ept2)],
                             row_v.at[pl.ds(ept1, ept2)], sem1),
            pltpu.async_copy(a1_hbm.at[pl.ds(wid * ept1, ept1)],
                             a_v.at[pl.ds(0, ept1)], sem2),
            pltpu.async_copy(a2_hbm.at[pl.ds(wid * ept2, ept2)],
                             a_v.at[pl.ds(ept1, ept2)], sem2),
        ]

        inf16 = jnp.full((_L,), jnp.inf, jnp.float32)

        def init_body(i, c):
            for t in tabs:
                t[pl.ds(i * _L, _L)] = inf16
            return c
        lax.fori_loop(0, npad // _L, init_body, 0)
        for cp in cps:
            cp.wait()

        trips = (ept1 + ept2) // (_L * _NT)

        def sweep(_):
            def body(i, acc):
                b = i * _NT * _L
                idxs = [row_v[pl.ds(b + u * _L, _L)] for u in range(_NT)]
                avs = [a_v[pl.ds(b + u * _L, _L)] for u in range(_NT)]
                curs = [plsc.load_gather(tabs[u], [idxs[u]])
                        for u in range(_NT)]
                losts = [avs[u] < curs[u] for u in range(_NT)]
                for u in range(_NT):
                    plsc.store_scatter(tabs[u], [idxs[u]], avs[u],
                                       mask=losts[u])
                chks = [plsc.load_gather(tabs[u], [idxs[u]])
                        for u in range(_NT)]
                for u in range(_NT):
                    acc = acc | (avs[u] < chks[u])
                return acc
            return lax.fori_loop(0, trips, body, jnp.zeros((_L,), jnp.bool_))

        fail = sweep(0)
        lax.while_loop(lambda f: jnp.any(f), sweep, fail)

        def merge_body(j, c):
            jo = j * _L
            m0 = tabs[0][pl.ds(jo, _L)]
            for t in tabs[1:]:
                m0 = jnp.minimum(m0, t[pl.ds(jo, _L)])
            tabm[pl.ds(jo, _L)] = m0
            return c
        lax.fori_loop(0, npad // _L, merge_body, 0)

        pltpu.sync_copy(tabm, sp_tab.at[sid])
        plsc.subcore_barrier()

        cb = sid * chunk
        stage = tabs[0]
        descs = [pltpu.async_copy(sp_tab.at[r, pl.ds(cb, chunk)],
                                  stage.at[pl.ds(r * chunk, chunk)], sem1)
                 for r in range(_NS)]
        for d in descs:
            d.wait()

        res = tabs[1]

        def red_body(j, c):
            jo = j * _L
            m0 = stage[pl.ds(jo, _L)]
            for r in range(1, _NS):
                m0 = jnp.minimum(m0, stage[pl.ds(r * chunk + jo, _L)])
            res[pl.ds(cb + jo, _L)] = m0
            return c
        lax.fori_loop(0, chunk // _L, red_body, 0)

        pltpu.sync_copy(res.at[pl.ds(cb, chunk)],
                        tab_hbm.at[pl.ds(cid * npad + cb, chunk)])

    @functools.partial(
        pl.kernel,
        out_type=jax.ShapeDtypeStruct((E,), jnp.float32),
        mesh=mesh,
        compiler_params=cparams,
        scratch_types=[
            pltpu.VMEM((_NC * npad,), jnp.float32),
            pltpu.VMEM((npad,), jnp.float32),
            pltpu.VMEM((ept1 + ept2,), jnp.int32),
            pltpu.VMEM((ept1 + ept2,), jnp.float32),
            pltpu.VMEM((ept1 + ept2,), jnp.float32),
            pltpu.SemaphoreType.DMA,
            pltpu.SemaphoreType.DMA,
        ],
    )
    def k2(tab_hbm, row_hbm, a1_hbm, a2_hbm, out_hbm, s2_v, tabm, row_v,
           a_v, w_v, sem1, sem2):
        cid = lax.axis_index("c")
        sid = lax.axis_index("s")
        wid = cid * _NS + sid

        cp_tab = pltpu.async_copy(tab_hbm, s2_v, sem1)
        cps = [
            pltpu.async_copy(row_hbm.at[pl.ds(wid * ept1, ept1)],
                             row_v.at[pl.ds(0, ept1)], sem2),
            pltpu.async_copy(row_hbm.at[pl.ds(e1 + wid * ept2, ept2)],
                             row_v.at[pl.ds(ept1, ept2)], sem2),
            pltpu.async_copy(a1_hbm.at[pl.ds(wid * ept1, ept1)],
                             a_v.at[pl.ds(0, ept1)], sem2),
            pltpu.async_copy(a2_hbm.at[pl.ds(wid * ept2, ept2)],
                             a_v.at[pl.ds(ept1, ept2)], sem2),
        ]
        cp_tab.wait()

        def merge_body(j, c):
            jo = j * _L
            tabm[pl.ds(jo, _L)] = jnp.minimum(s2_v[pl.ds(jo, _L)],
                                              s2_v[pl.ds(npad + jo, _L)])
            return c
        lax.fori_loop(0, npad // _L, merge_body, 0)
        for cp in cps:
            cp.wait()

        def p2_body(j, c):
            b = j * _NT * _L
            idxs = [row_v[pl.ds(b + u * _L, _L)] for u in range(_NT)]
            avs = [a_v[pl.ds(b + u * _L, _L)] for u in range(_NT)]
            ms = [plsc.load_gather(tabm, [idxs[u]]) for u in range(_NT)]
            for u in range(_NT):
                w_v[pl.ds(b + u * _L, _L)] = jnp.maximum(
                    avs[u] / ms[u] - _THETA, 0.0)
            return c
        lax.fori_loop(0, (ept1 + ept2) // (_L * _NT), p2_body, 0)

        pltpu.sync_copy(w_v.at[pl.ds(0, ept1)],
                        out_hbm.at[pl.ds(wid * ept1, ept1)])
        pltpu.sync_copy(w_v.at[pl.ds(ept1, ept2)],
                        out_hbm.at[pl.ds(e1 + wid * ept2, ept2)])

    tab2 = k1(row, a1, a2)
    return k2(tab2, row, a1, a2)


def kernel(vertex_attr, edgeij_pair, edge_attr):
    return _segmin_edge_update(edgeij_pair, edge_attr, vertex_attr.shape[0])


# speedup vs baseline: 1.1627x; 1.0223x over previous
import functools

import jax
import jax.numpy as jnp
from jax import lax
from jax.experimental import pallas as pl
from jax.experimental.pallas import tpu as pltpu
from jax.experimental.pallas import tpu_sc as plsc

_THETA = 0.25
_L = 16
_NC = 2
_NS = 16
_NT = 5
_NW = _NC * _NS


def _row0_body(pair_ref, out_ref):
    out_ref[...] = pair_ref[0]


def _extract_row0(edgeij_pair):
    E = edgeij_pair.shape[1]
    return pl.pallas_call(
        _row0_body,
        out_shape=jax.ShapeDtypeStruct((E,), edgeij_pair.dtype),
    )(edgeij_pair)


@functools.partial(jax.jit, static_argnums=(2,))
def _segmin_edge_update(pair, edge_attr, n_nodes):
    row = _extract_row0(pair)
    E = pair.shape[1]
    grain = _L * _NT * _NW
    e1 = ((E // 2) // grain) * grain + grain
    e2 = E - e1
    assert e2 % grain == 0
    ept1 = e1 // _NW
    ept2 = e2 // _NW
    npad = ((n_nodes + _L * _NS - 1) // (_L * _NS)) * (_L * _NS)
    chunk = npad // _NS

    a1 = edge_attr[:e1, 0]
    a2 = edge_attr[e1:, 0]

    mesh = plsc.VectorSubcoreMesh(core_axis_name="c", subcore_axis_name="s")
    cparams = pltpu.CompilerParams(needs_layout_passes=False)

    @functools.partial(
        pl.kernel,
        out_type=jax.ShapeDtypeStruct((_NC * npad,), jnp.float32),
        mesh=mesh,
        compiler_params=cparams,
        scratch_types=[
            pltpu.VMEM((ept1 + ept2,), jnp.int32),
            pltpu.VMEM((ept1 + ept2,), jnp.float32),
            [pltpu.VMEM((npad,), jnp.float32) for _ in range(_NT)],
            pltpu.VMEM((npad,), jnp.float32),
            pltpu.VMEM_SHARED((_NS, npad), jnp.float32),
            pltpu.SemaphoreType.DMA,
            pltpu.SemaphoreType.DMA,
        ],
    )
    def k1(row_hbm, a1_hbm, a2_hbm, tab_hbm, row_v, a_v, tabs, tabm,
           sp_tab, sem1, sem2):
        cid = lax.axis_index("c")
        sid = lax.axis_index("s")
        wid = cid * _NS + sid

        cps = [
            pltpu.async_copy(row_hbm.at[pl.ds(wid * ept1, ept1)],
                             row_v.at[pl.ds(0, ept1)], sem1),
            pltpu.async_copy(row_hbm.at[pl.ds(e1 + wid * ept2, ept2)],
                             row_v.at[pl.ds(ept1, ept2)], sem1),
            pltpu.async_copy(a1_hbm.at[pl.ds(wid * ept1, ept1)],
                             a_v.at[pl.ds(0, ept1)], sem2),
            pltpu.async_copy(a2_hbm.at[pl.ds(wid * ept2, ept2)],
                             a_v.at[pl.ds(ept1, ept2)], sem2),
        ]

        inf16 = jnp.full((_L,), jnp.inf, jnp.float32)

        def init_body(i, c):
            for t in tabs:
                t[pl.ds(i * _L, _L)] = inf16
            return c
        lax.fori_loop(0, npad // _L, init_body, 0)
        for cp in cps:
            cp.wait()

        trips = (ept1 + ept2) // (_L * _NT)

        def p1_body(i, c):
            b = i * _NT * _L
            idxs = [row_v[pl.ds(b + u * _L, _L)] for u in range(_NT)]
            avs = [a_v[pl.ds(b + u * _L, _L)] for u in range(_NT)]
            curs = [plsc.load_gather(tabs[u], [idxs[u]])
                    for u in range(_NT)]
            losts = [avs[u] < curs[u] for u in range(_NT)]
            for u in range(_NT):
                plsc.store_scatter(tabs[u], [idxs[u]], avs[u],
                                   mask=losts[u])
            chks = [plsc.load_gather(tabs[u], [idxs[u]])
                    for u in range(_NT)]
            fails = [avs[u] < chks[u] for u in range(_NT)]
            anyfail = fails[0]
            for f in fails[1:]:
                anyfail = anyfail | f

            @pl.when(jnp.any(anyfail))
            def _fix():
                for u in range(_NT):
                    def wcond(m):
                        return jnp.any(m)

                    def wbody(m, u=u):
                        plsc.store_scatter(tabs[u], [idxs[u]], avs[u],
                                           mask=m)
                        chk2 = plsc.load_gather(tabs[u], [idxs[u]])
                        return m & (avs[u] < chk2)
                    lax.while_loop(wcond, wbody, fails[u])
            return c
        lax.fori_loop(0, trips, p1_body, 0)

        def merge_body(j, c):
            jo = j * _L
            m0 = tabs[0][pl.ds(jo, _L)]
            for t in tabs[1:]:
                m0 = jnp.minimum(m0, t[pl.ds(jo, _L)])
            tabm[pl.ds(jo, _L)] = m0
            return c
        lax.fori_loop(0, npad // _L, merge_body, 0)

        pltpu.sync_copy(tabm, sp_tab.at[sid])
        plsc.subcore_barrier()

        cb = sid * chunk
        stage = tabs[0]
        descs = [pltpu.async_copy(sp_tab.at[r, pl.ds(cb, chunk)],
                                  stage.at[pl.ds(r * chunk, chunk)], sem1)
                 for r in range(_NS)]
        for d in descs:
            d.wait()

        res = tabs[1]

        def red_body(j, c):
            jo = j * _L
            m0 = stage[pl.ds(jo, _L)]
            for r in range(1, _NS):
                m0 = jnp.minimum(m0, stage[pl.ds(r * chunk + jo, _L)])
            res[pl.ds(cb + jo, _L)] = m0
            return c
        lax.fori_loop(0, chunk // _L, red_body, 0)

        pltpu.sync_copy(res.at[pl.ds(cb, chunk)],
                        tab_hbm.at[pl.ds(cid * npad + cb, chunk)])

    @functools.partial(
        pl.kernel,
        out_type=jax.ShapeDtypeStruct((E,), jnp.float32),
        mesh=mesh,
        compiler_params=cparams,
        scratch_types=[
            pltpu.VMEM((_NC * npad,), jnp.float32),
            pltpu.VMEM((npad,), jnp.float32),
            pltpu.VMEM((ept1 + ept2,), jnp.int32),
            pltpu.VMEM((ept1 + ept2,), jnp.float32),
            pltpu.VMEM((ept1 + ept2,), jnp.float32),
            pltpu.SemaphoreType.DMA,
            pltpu.SemaphoreType.DMA,
        ],
    )
    def k2(tab_hbm, row_hbm, a1_hbm, a2_hbm, out_hbm, s2_v, tabm, row_v,
           a_v, w_v, sem1, sem2):
        cid = lax.axis_index("c")
        sid = lax.axis_index("s")
        wid = cid * _NS + sid

        cp_tab = pltpu.async_copy(tab_hbm, s2_v, sem1)
        cps = [
            pltpu.async_copy(row_hbm.at[pl.ds(wid * ept1, ept1)],
                             row_v.at[pl.ds(0, ept1)], sem2),
            pltpu.async_copy(row_hbm.at[pl.ds(e1 + wid * ept2, ept2)],
                             row_v.at[pl.ds(ept1, ept2)], sem2),
            pltpu.async_copy(a1_hbm.at[pl.ds(wid * ept1, ept1)],
                             a_v.at[pl.ds(0, ept1)], sem2),
            pltpu.async_copy(a2_hbm.at[pl.ds(wid * ept2, ept2)],
                             a_v.at[pl.ds(ept1, ept2)], sem2),
        ]
        cp_tab.wait()

        def merge_body(j, c):
            jo = j * _L
            tabm[pl.ds(jo, _L)] = jnp.minimum(s2_v[pl.ds(jo, _L)],
                                              s2_v[pl.ds(npad + jo, _L)])
            return c
        lax.fori_loop(0, npad // _L, merge_body, 0)
        for cp in cps:
            cp.wait()

        def p2_body(j, c):
            b = j * _NT * _L
            idxs = [row_v[pl.ds(b + u * _L, _L)] for u in range(_NT)]
            avs = [a_v[pl.ds(b + u * _L, _L)] for u in range(_NT)]
            ms = [plsc.load_gather(tabm, [idxs[u]]) for u in range(_NT)]
            for u in range(_NT):
                w_v[pl.ds(b + u * _L, _L)] = jnp.maximum(
                    avs[u] / ms[u] - _THETA, 0.0)
            return c
        lax.fori_loop(0, (ept1 + ept2) // (_L * _NT), p2_body, 0)

        pltpu.sync_copy(w_v.at[pl.ds(0, ept1)],
                        out_hbm.at[pl.ds(wid * ept1, ept1)])
        pltpu.sync_copy(w_v.at[pl.ds(ept1, ept2)],
                        out_hbm.at[pl.ds(e1 + wid * ept2, ept2)])

    tab2 = k1(row, a1, a2)
    return k2(tab2, row, a1, a2)


def kernel(vertex_attr, edgeij_pair, edge_attr):
    return _segmin_edge_update(edgeij_pair, edge_attr, vertex_attr.shape[0])


# speedup vs baseline: 1.2104x; 1.0410x over previous
import functools

import jax
import jax.numpy as jnp
from jax import lax
from jax.experimental import pallas as pl
from jax.experimental.pallas import tpu as pltpu
from jax.experimental.pallas import tpu_sc as plsc

_THETA = 0.25
_L = 16
_NC = 2
_NS = 16
_NT = 5
_NW = _NC * _NS


def _row0_body(pair_ref, out_ref):
    out_ref[...] = pair_ref[0]


def _extract_row0(edgeij_pair):
    E = edgeij_pair.shape[1]
    return pl.pallas_call(
        _row0_body,
        out_shape=jax.ShapeDtypeStruct((E,), edgeij_pair.dtype),
    )(edgeij_pair)


@functools.partial(jax.jit, static_argnums=(2,))
def _segmin_edge_update(pair, edge_attr, n_nodes):
    row = _extract_row0(pair)
    E = pair.shape[1]
    grain = _L * _NT * _NW
    e1 = ((E // 2) // grain) * grain + grain
    e2 = E - e1
    assert e2 % grain == 0
    ept1 = e1 // _NW
    ept2 = e2 // _NW
    npad = ((n_nodes + _L * _NS - 1) // (_L * _NS)) * (_L * _NS)
    chunk = npad // _NS

    a1 = edge_attr[:e1, 0]
    a2 = edge_attr[e1:, 0]

    mesh = plsc.VectorSubcoreMesh(core_axis_name="c", subcore_axis_name="s")
    cparams = pltpu.CompilerParams(needs_layout_passes=False)

    @functools.partial(
        pl.kernel,
        out_type=jax.ShapeDtypeStruct((_NC * npad,), jnp.float32),
        mesh=mesh,
        compiler_params=cparams,
        scratch_types=[
            pltpu.VMEM((ept1 + ept2,), jnp.int32),
            pltpu.VMEM((ept1 + ept2,), jnp.float32),
            [pltpu.VMEM((npad,), jnp.float32) for _ in range(_NT)],
            pltpu.VMEM((npad,), jnp.float32),
            pltpu.VMEM_SHARED((_NS, npad), jnp.float32),
            pltpu.SemaphoreType.DMA,
            pltpu.SemaphoreType.DMA,
        ],
    )
    def k1(row_hbm, a1_hbm, a2_hbm, tab_hbm, row_v, a_v, tabs, tabm,
           sp_tab, sem1, sem2):
        cid = lax.axis_index("c")
        sid = lax.axis_index("s")
        wid = cid * _NS + sid

        cps = [
            pltpu.async_copy(row_hbm.at[pl.ds(wid * ept1, ept1)],
                             row_v.at[pl.ds(0, ept1)], sem1),
            pltpu.async_copy(row_hbm.at[pl.ds(e1 + wid * ept2, ept2)],
                             row_v.at[pl.ds(ept1, ept2)], sem1),
            pltpu.async_copy(a1_hbm.at[pl.ds(wid * ept1, ept1)],
                             a_v.at[pl.ds(0, ept1)], sem2),
            pltpu.async_copy(a2_hbm.at[pl.ds(wid * ept2, ept2)],
                             a_v.at[pl.ds(ept1, ept2)], sem2),
        ]

        inf16 = jnp.full((_L,), jnp.inf, jnp.float32)

        def init_body(i, c):
            for t in tabs:
                t[pl.ds(i * _L, _L)] = inf16
            return c
        lax.fori_loop(0, npad // _L, init_body, 0)
        for cp in cps:
            cp.wait()

        trips = (ept1 + ept2) // (_L * _NT)

        def p1_body(i, c):
            b = i * _NT * _L
            idxs = [row_v[pl.ds(b + u * _L, _L)] for u in range(_NT)]
            avs = [a_v[pl.ds(b + u * _L, _L)] for u in range(_NT)]
            curs = [plsc.load_gather(tabs[u], [idxs[u]])
                    for u in range(_NT)]
            losts = [avs[u] < curs[u] for u in range(_NT)]
            for u in range(_NT):
                plsc.store_scatter(tabs[u], [idxs[u]], avs[u],
                                   mask=losts[u])
            chks = [plsc.load_gather(tabs[u], [idxs[u]])
                    for u in range(_NT)]
            fails = [avs[u] < chks[u] for u in range(_NT)]
            anyfail = fails[0]
            for f in fails[1:]:
                anyfail = anyfail | f

            @pl.when(jnp.any(anyfail))
            def _fix():
                for u in range(_NT):
                    def wcond(m):
                        return jnp.any(m)

                    def wbody(m, u=u):
                        plsc.store_scatter(tabs[u], [idxs[u]], avs[u],
                                           mask=m)
                        chk2 = plsc.load_gather(tabs[u], [idxs[u]])
                        return m & (avs[u] < chk2)
                    lax.while_loop(wcond, wbody, fails[u])
            return c
        lax.fori_loop(0, trips, p1_body, 0)

        def merge_body(j, c):
            jo = j * _L
            m0 = tabs[0][pl.ds(jo, _L)]
            for t in tabs[1:]:
                m0 = jnp.minimum(m0, t[pl.ds(jo, _L)])
            tabm[pl.ds(jo, _L)] = m0
            return c
        lax.fori_loop(0, npad // _L, merge_body, 0)

        pltpu.sync_copy(tabm, sp_tab.at[sid])
        plsc.subcore_barrier()

        cb = sid * chunk
        stage = tabs[0]
        descs = [pltpu.async_copy(sp_tab.at[r, pl.ds(cb, chunk)],
                                  stage.at[pl.ds(r * chunk, chunk)], sem1)
                 for r in range(_NS)]
        for d in descs:
            d.wait()

        res = tabs[1]

        def red_body(j, c):
            jo = j * _L
            m0 = stage[pl.ds(jo, _L)]
            for r in range(1, _NS):
                m0 = jnp.minimum(m0, stage[pl.ds(r * chunk + jo, _L)])
            res[pl.ds(cb + jo, _L)] = m0
            return c
        lax.fori_loop(0, chunk // _L, red_body, 0)

        pltpu.sync_copy(res.at[pl.ds(cb, chunk)],
                        tab_hbm.at[pl.ds(cid * npad + cb, chunk)])

    @functools.partial(
        pl.kernel,
        out_type=jax.ShapeDtypeStruct((E,), jnp.float32),
        mesh=mesh,
        compiler_params=cparams,
        scratch_types=[
            [pltpu.VMEM((npad,), jnp.float32) for _ in range(_NC)],
            pltpu.VMEM((ept1 + ept2,), jnp.int32),
            pltpu.VMEM((ept1 + ept2,), jnp.float32),
            pltpu.VMEM((ept1 + ept2,), jnp.float32),
            pltpu.SemaphoreType.DMA,
            pltpu.SemaphoreType.DMA,
        ],
    )
    def k2(tab_hbm, row_hbm, a1_hbm, a2_hbm, out_hbm, halves, row_v,
           a_v, w_v, sem1, sem2):
        cid = lax.axis_index("c")
        sid = lax.axis_index("s")
        wid = cid * _NS + sid

        cp_tabs = [pltpu.async_copy(tab_hbm.at[pl.ds(h * npad, npad)],
                                    halves[h], sem1) for h in range(_NC)]
        cps = [
            pltpu.async_copy(row_hbm.at[pl.ds(wid * ept1, ept1)],
                             row_v.at[pl.ds(0, ept1)], sem2),
            pltpu.async_copy(row_hbm.at[pl.ds(e1 + wid * ept2, ept2)],
                             row_v.at[pl.ds(ept1, ept2)], sem2),
            pltpu.async_copy(a1_hbm.at[pl.ds(wid * ept1, ept1)],
                             a_v.at[pl.ds(0, ept1)], sem2),
            pltpu.async_copy(a2_hbm.at[pl.ds(wid * ept2, ept2)],
                             a_v.at[pl.ds(ept1, ept2)], sem2),
        ]
        for cp in cp_tabs:
            cp.wait()
        for cp in cps:
            cp.wait()

        def p2_body(j, c):
            b = j * _NT * _L
            idxs = [row_v[pl.ds(b + u * _L, _L)] for u in range(_NT)]
            avs = [a_v[pl.ds(b + u * _L, _L)] for u in range(_NT)]
            m0s = [plsc.load_gather(halves[0], [idxs[u]])
                   for u in range(_NT)]
            m1s = [plsc.load_gather(halves[1], [idxs[u]])
                   for u in range(_NT)]
            for u in range(_NT):
                m = jnp.minimum(m0s[u], m1s[u])
                w_v[pl.ds(b + u * _L, _L)] = jnp.maximum(
                    avs[u] / m - _THETA, 0.0)
            return c
        lax.fori_loop(0, (ept1 + ept2) // (_L * _NT), p2_body, 0)

        pltpu.sync_copy(w_v.at[pl.ds(0, ept1)],
                        out_hbm.at[pl.ds(wid * ept1, ept1)])
        pltpu.sync_copy(w_v.at[pl.ds(ept1, ept2)],
                        out_hbm.at[pl.ds(e1 + wid * ept2, ept2)])

    tab2 = k1(row, a1, a2)
    return k2(tab2, row, a1, a2)


def kernel(vertex_attr, edgeij_pair, edge_attr):
    return _segmin_edge_update(edgeij_pair, edge_attr, vertex_attr.shape[0])
